# baseline (device time: 102334 ns/iter reference)
import jax
import jax.numpy as jnp
from jax import lax
from jax.experimental import pallas as pl
from jax.experimental.pallas import tpu as pltpu

N_DEV = 4


def kernel(x, W):
    t, d = x.shape
    _, v_loc = W.shape
    v_half = v_loc // 2

    def body(x_ref, w_ref, out_ref, comm_r, comm_l, stats_buf, stg,
             send_r, recv_r, send_l, recv_l,
             stats_send, stats_recv, stg_sems):
        my_pos = lax.axis_index("i")
        left = (my_pos - 1) % N_DEV
        right = (my_pos + 1) % N_DEV
        diag = (my_pos + 2) % N_DEV

        barrier_sem = pltpu.get_barrier_semaphore()
        for nbr in [left, right, diag]:
            pl.semaphore_signal(
                barrier_sem, inc=1,
                device_id=(nbr,), device_id_type=pl.DeviceIdType.MESH,
            )
        pl.semaphore_wait(barrier_sem, N_DEV - 1)

        def mk_ring(comm, sends, recvs, h, dev):
            return pltpu.make_async_remote_copy(
                src_ref=comm.at[h], dst_ref=comm.at[h + 1],
                send_sem=sends.at[h], recv_sem=recvs.at[h],
                device_id=(dev,), device_id_type=pl.DeviceIdType.MESH,
            )

        logits = jnp.dot(x_ref[:, :], w_ref[:, :],
                         preferred_element_type=jnp.float32)
        comm_r[0, :, :] = logits[:, :v_half]
        comm_l[0, :, :] = logits[:, v_half:]
        ring_r = [mk_ring(comm_r, send_r, recv_r, 0, right)]
        ring_l = [mk_ring(comm_l, send_l, recv_l, 0, left)]
        ring_r[0].start()
        ring_l[0].start()


        m0 = jnp.max(logits, axis=-1, keepdims=True)
        e0 = jnp.exp(logits - m0)
        s0 = jnp.sum(e0, axis=-1, keepdims=True)
        stats_buf[my_pos, 0, :, :] = jnp.broadcast_to(m0, (t, 128))
        stats_buf[my_pos, 1, :, :] = jnp.broadcast_to(s0, (t, 128))
        stats_rdmas = []
        for k, p in enumerate([left, right, diag]):
            r = pltpu.make_async_remote_copy(
                src_ref=stats_buf.at[my_pos], dst_ref=stats_buf.at[my_pos],
                send_sem=stats_send.at[k], recv_sem=stats_recv.at[my_pos],
                device_id=(p,), device_id_type=pl.DeviceIdType.MESH,
            )
            r.start()
            stats_rdmas.append(r)
        for p in [left, right, diag]:
            pltpu.make_async_remote_copy(
                src_ref=stats_buf.at[p], dst_ref=stats_buf.at[p],
                send_sem=stats_send.at[0], recv_sem=stats_recv.at[p],
                device_id=(p,), device_id_type=pl.DeviceIdType.MESH,
            ).wait_recv()
        for r in stats_rdmas:
            r.wait_send()

        ms = [stats_buf[c, 0, :, :] for c in range(N_DEV)]
        ss = [stats_buf[c, 1, :, :] for c in range(N_DEV)]
        m = ms[0]
        for mc in ms[1:]:
            m = jnp.maximum(m, mc)
        z = ss[0] * jnp.exp(ms[0] - m)
        for mc, sc in zip(ms[1:], ss[1:]):
            z = z + sc * jnp.exp(mc - m)
        mcol = m[:, 0:1]
        inv_z = 1.0 / z[:, 0:1]

        dmas = {}

        def emit(ping, j, piece_final, start):
            stg[ping, j, :, :] = piece_final
            dma = pltpu.make_async_copy(
                stg.at[ping, j],
                out_ref.at[:, pl.ds(start, v_half)],
                stg_sems.at[ping * 2 + j],
            )
            dma.start()
            dmas[(ping, j)] = dma

        own_scale = jnp.exp(m0 - mcol) * inv_z
        own = e0 * own_scale
        emit(0, 0, own[:, :v_half], my_pos * v_loc)
        emit(0, 1, own[:, v_half:], my_pos * v_loc + v_half)

        for h in range(N_DEV - 1):
            ping = (h + 1) % 2
            ring_r[h].wait_recv()
            if h < N_DEV - 2:
                ring_r.append(mk_ring(comm_r, send_r, recv_r, h + 1, right))
                ring_r[h + 1].start()
            ring_l[h].wait_recv()
            if h < N_DEV - 2:
                ring_l.append(mk_ring(comm_l, send_l, recv_l, h + 1, left))
                ring_l[h + 1].start()
            ring_r[h].wait_send()
            ring_l[h].wait_send()

            for j in (0, 1):
                if (ping, j) in dmas:
                    dmas[(ping, j)].wait()

            origin_r = (my_pos - h - 1) % N_DEV
            origin_l = (my_pos + h + 1) % N_DEV
            fin_r = jnp.exp(comm_r[h + 1, :, :] - mcol) * inv_z
            emit(ping, 0, fin_r, origin_r * v_loc)
            fin_l = jnp.exp(comm_l[h + 1, :, :] - mcol) * inv_z
            emit(ping, 1, fin_l, origin_l * v_loc + v_half)

        for ping in (0, 1):
            for j in (0, 1):
                if (ping, j) in dmas:
                    dmas[(ping, j)].wait()

    return pl.pallas_call(
        body,
        out_shape=jax.ShapeDtypeStruct((t, N_DEV * v_loc), jnp.float32),
        in_specs=[
            pl.BlockSpec(memory_space=pltpu.VMEM),
            pl.BlockSpec(memory_space=pltpu.VMEM),
        ],
        out_specs=pl.BlockSpec(memory_space=pl.ANY),
        scratch_shapes=[
            pltpu.VMEM((N_DEV, t, v_half), jnp.float32),
            pltpu.VMEM((N_DEV, t, v_half), jnp.float32),
            pltpu.VMEM((N_DEV, 2, t, 128), jnp.float32),
            pltpu.VMEM((2, 2, t, v_half), jnp.float32),
            pltpu.SemaphoreType.DMA((N_DEV - 1,)),
            pltpu.SemaphoreType.DMA((N_DEV - 1,)),
            pltpu.SemaphoreType.DMA((N_DEV - 1,)),
            pltpu.SemaphoreType.DMA((N_DEV - 1,)),
            pltpu.SemaphoreType.DMA((3,)),
            pltpu.SemaphoreType.DMA((N_DEV,)),
            pltpu.SemaphoreType.DMA((4,)),
        ],
        compiler_params=pltpu.CompilerParams(
            collective_id=0,
            vmem_limit_bytes=100 * 1024 * 1024,
        ),
    )(x, W)


# device time: 101172 ns/iter; 1.0115x vs baseline; 1.0115x over previous
import jax
import jax.numpy as jnp
from jax import lax
from jax.experimental import pallas as pl
from jax.experimental.pallas import tpu as pltpu

N_DEV = 4


def kernel(x, W):
    t, d = x.shape
    _, v_loc = W.shape
    v_half = v_loc // 2

    def body(x_hbm, w_hbm, out_ref, comm_r, comm_l, stats_buf, stg,
             x_v, w_v,
             send_r, recv_r, send_l, recv_l,
             stats_send, stats_recv, stg_sems, in_sems):
        my_pos = lax.axis_index("i")
        left = (my_pos - 1) % N_DEV
        right = (my_pos + 1) % N_DEV
        diag = (my_pos + 2) % N_DEV

        dma_x = pltpu.make_async_copy(x_hbm, x_v, in_sems.at[0])
        dma_w = pltpu.make_async_copy(w_hbm, w_v, in_sems.at[1])
        dma_x.start()
        dma_w.start()

        barrier_sem = pltpu.get_barrier_semaphore()
        for nbr in [left, right, diag]:
            pl.semaphore_signal(
                barrier_sem, inc=1,
                device_id=(nbr,), device_id_type=pl.DeviceIdType.MESH,
            )
        pl.semaphore_wait(barrier_sem, N_DEV - 1)

        def mk_ring(comm, sends, recvs, h, dev):
            return pltpu.make_async_remote_copy(
                src_ref=comm.at[h], dst_ref=comm.at[h + 1],
                send_sem=sends.at[h], recv_sem=recvs.at[h],
                device_id=(dev,), device_id_type=pl.DeviceIdType.MESH,
            )

        dma_x.wait()
        dma_w.wait()
        logits = jnp.dot(x_v[:, :], w_v[:, :],
                         preferred_element_type=jnp.float32)
        comm_r[0, :, :] = logits[:, :v_half]
        comm_l[0, :, :] = logits[:, v_half:]
        ring_r = [mk_ring(comm_r, send_r, recv_r, 0, right)]
        ring_l = [mk_ring(comm_l, send_l, recv_l, 0, left)]
        ring_r[0].start()
        ring_l[0].start()


        m0 = jnp.max(logits, axis=-1, keepdims=True)
        e0 = jnp.exp(logits - m0)
        s0 = jnp.sum(e0, axis=-1, keepdims=True)
        stats_buf[my_pos, 0, :, :] = jnp.broadcast_to(m0, (t, 128))
        stats_buf[my_pos, 1, :, :] = jnp.broadcast_to(s0, (t, 128))
        stats_rdmas = []
        for k, p in enumerate([left, right, diag]):
            r = pltpu.make_async_remote_copy(
                src_ref=stats_buf.at[my_pos], dst_ref=stats_buf.at[my_pos],
                send_sem=stats_send.at[k], recv_sem=stats_recv.at[my_pos],
                device_id=(p,), device_id_type=pl.DeviceIdType.MESH,
            )
            r.start()
            stats_rdmas.append(r)
        for p in [left, right, diag]:
            pltpu.make_async_remote_copy(
                src_ref=stats_buf.at[p], dst_ref=stats_buf.at[p],
                send_sem=stats_send.at[0], recv_sem=stats_recv.at[p],
                device_id=(p,), device_id_type=pl.DeviceIdType.MESH,
            ).wait_recv()
        for r in stats_rdmas:
            r.wait_send()

        ms = [stats_buf[c, 0, :, :] for c in range(N_DEV)]
        ss = [stats_buf[c, 1, :, :] for c in range(N_DEV)]
        m = ms[0]
        for mc in ms[1:]:
            m = jnp.maximum(m, mc)
        z = ss[0] * jnp.exp(ms[0] - m)
        for mc, sc in zip(ms[1:], ss[1:]):
            z = z + sc * jnp.exp(mc - m)
        mcol = m[:, 0:1]
        inv_z = 1.0 / z[:, 0:1]

        dmas = {}

        def emit(ping, j, piece_final, start):
            stg[ping, j, :, :] = piece_final
            dma = pltpu.make_async_copy(
                stg.at[ping, j],
                out_ref.at[:, pl.ds(start, v_half)],
                stg_sems.at[ping * 2 + j],
            )
            dma.start()
            dmas[(ping, j)] = dma

        own_scale = jnp.exp(m0 - mcol) * inv_z
        own = e0 * own_scale
        emit(0, 0, own[:, :v_half], my_pos * v_loc)
        emit(0, 1, own[:, v_half:], my_pos * v_loc + v_half)

        for h in range(N_DEV - 1):
            ping = (h + 1) % 2
            ring_r[h].wait_recv()
            if h < N_DEV - 2:
                ring_r.append(mk_ring(comm_r, send_r, recv_r, h + 1, right))
                ring_r[h + 1].start()
            ring_l[h].wait_recv()
            if h < N_DEV - 2:
                ring_l.append(mk_ring(comm_l, send_l, recv_l, h + 1, left))
                ring_l[h + 1].start()
            ring_r[h].wait_send()
            ring_l[h].wait_send()

            for j in (0, 1):
                if (ping, j) in dmas:
                    dmas[(ping, j)].wait()

            origin_r = (my_pos - h - 1) % N_DEV
            origin_l = (my_pos + h + 1) % N_DEV
            fin_r = jnp.exp(comm_r[h + 1, :, :] - mcol) * inv_z
            emit(ping, 0, fin_r, origin_r * v_loc)
            fin_l = jnp.exp(comm_l[h + 1, :, :] - mcol) * inv_z
            emit(ping, 1, fin_l, origin_l * v_loc + v_half)

        for ping in (0, 1):
            for j in (0, 1):
                if (ping, j) in dmas:
                    dmas[(ping, j)].wait()

    return pl.pallas_call(
        body,
        out_shape=jax.ShapeDtypeStruct((t, N_DEV * v_loc), jnp.float32),
        in_specs=[
            pl.BlockSpec(memory_space=pl.ANY),
            pl.BlockSpec(memory_space=pl.ANY),
        ],
        out_specs=pl.BlockSpec(memory_space=pl.ANY),
        scratch_shapes=[
            pltpu.VMEM((N_DEV, t, v_half), jnp.float32),
            pltpu.VMEM((N_DEV, t, v_half), jnp.float32),
            pltpu.VMEM((N_DEV, 2, t, 128), jnp.float32),
            pltpu.VMEM((2, 2, t, v_half), jnp.float32),
            pltpu.VMEM((t, d), jnp.float32),
            pltpu.VMEM((d, v_loc), jnp.float32),
            pltpu.SemaphoreType.DMA((N_DEV - 1,)),
            pltpu.SemaphoreType.DMA((N_DEV - 1,)),
            pltpu.SemaphoreType.DMA((N_DEV - 1,)),
            pltpu.SemaphoreType.DMA((N_DEV - 1,)),
            pltpu.SemaphoreType.DMA((3,)),
            pltpu.SemaphoreType.DMA((N_DEV,)),
            pltpu.SemaphoreType.DMA((4,)),
            pltpu.SemaphoreType.DMA((2,)),
        ],
        compiler_params=pltpu.CompilerParams(
            collective_id=0,
            vmem_limit_bytes=100 * 1024 * 1024,
        ),
    )(x, W)


# device time: 66879 ns/iter; 1.5301x vs baseline; 1.5128x over previous
import jax
import jax.numpy as jnp
from jax import lax
from jax.experimental import pallas as pl
from jax.experimental.pallas import tpu as pltpu

N_DEV = 4


def kernel(x, W):
    t, d = x.shape
    _, v_loc = W.shape
    v_half = v_loc // 2

    def body(x_hbm, w_hbm, out_ref, comm_r, comm_l, stats_buf, stg,
             x_v, w_v,
             send_r, recv_r, send_l, recv_l,
             stats_send, stats_recv, stg_sems, in_sems):
        my_pos = lax.axis_index("i")
        left = (my_pos - 1) % N_DEV
        right = (my_pos + 1) % N_DEV
        diag = (my_pos + 2) % N_DEV

        dma_x = pltpu.make_async_copy(x_hbm, x_v, in_sems.at[0])
        dma_wa = pltpu.make_async_copy(
            w_hbm.at[:, pl.ds(0, v_half)],
            w_v.at[:, pl.ds(0, v_half)], in_sems.at[1])
        dma_wb = pltpu.make_async_copy(
            w_hbm.at[:, pl.ds(v_half, v_half)],
            w_v.at[:, pl.ds(v_half, v_half)], in_sems.at[2])
        dma_x.start()
        dma_wa.start()
        dma_wb.start()

        barrier_sem = pltpu.get_barrier_semaphore()
        for nbr in [left, right, diag]:
            pl.semaphore_signal(
                barrier_sem, inc=1,
                device_id=(nbr,), device_id_type=pl.DeviceIdType.MESH,
            )
        pl.semaphore_wait(barrier_sem, N_DEV - 1)

        def mk_ring(comm, sends, recvs, h, dev):
            return pltpu.make_async_remote_copy(
                src_ref=comm.at[h], dst_ref=comm.at[h + 1],
                send_sem=sends.at[h], recv_sem=recvs.at[h],
                device_id=(dev,), device_id_type=pl.DeviceIdType.MESH,
            )

        dma_x.wait()
        dma_wa.wait()
        logits_a = jnp.dot(x_v[:, :], w_v[:, pl.ds(0, v_half)],
                           preferred_element_type=jnp.float32)
        comm_r[0, :, :] = logits_a.astype(jnp.bfloat16)
        ring_r = [mk_ring(comm_r, send_r, recv_r, 0, right)]
        ring_r[0].start()

        dma_wb.wait()
        logits_b = jnp.dot(x_v[:, :], w_v[:, pl.ds(v_half, v_half)],
                           preferred_element_type=jnp.float32)
        comm_l[0, :, :] = logits_b.astype(jnp.bfloat16)
        ring_l = [mk_ring(comm_l, send_l, recv_l, 0, left)]
        ring_l[0].start()


        ma = jnp.max(logits_a, axis=-1, keepdims=True)
        mb = jnp.max(logits_b, axis=-1, keepdims=True)
        m0 = jnp.maximum(ma, mb)
        ea = jnp.exp(logits_a - m0)
        eb = jnp.exp(logits_b - m0)
        s0 = (jnp.sum(ea, axis=-1, keepdims=True)
              + jnp.sum(eb, axis=-1, keepdims=True))
        stats_buf[my_pos, 0, :, :] = jnp.broadcast_to(m0, (t, 128))
        stats_buf[my_pos, 1, :, :] = jnp.broadcast_to(s0, (t, 128))
        stats_rdmas = []
        for k, p in enumerate([left, right, diag]):
            r = pltpu.make_async_remote_copy(
                src_ref=stats_buf.at[my_pos], dst_ref=stats_buf.at[my_pos],
                send_sem=stats_send.at[k], recv_sem=stats_recv.at[my_pos],
                device_id=(p,), device_id_type=pl.DeviceIdType.MESH,
            )
            r.start()
            stats_rdmas.append(r)
        for p in [left, right, diag]:
            pltpu.make_async_remote_copy(
                src_ref=stats_buf.at[p], dst_ref=stats_buf.at[p],
                send_sem=stats_send.at[0], recv_sem=stats_recv.at[p],
                device_id=(p,), device_id_type=pl.DeviceIdType.MESH,
            ).wait_recv()
        for r in stats_rdmas:
            r.wait_send()

        ms = [stats_buf[c, 0, :, :] for c in range(N_DEV)]
        ss = [stats_buf[c, 1, :, :] for c in range(N_DEV)]
        m = ms[0]
        for mc in ms[1:]:
            m = jnp.maximum(m, mc)
        z = ss[0] * jnp.exp(ms[0] - m)
        for mc, sc in zip(ms[1:], ss[1:]):
            z = z + sc * jnp.exp(mc - m)
        mcol = m[:, 0:1]
        inv_z = 1.0 / z[:, 0:1]

        dmas = {}

        def emit(ping, j, piece_final, start):
            stg[ping, j, :, :] = piece_final
            dma = pltpu.make_async_copy(
                stg.at[ping, j],
                out_ref.at[:, pl.ds(start, v_half)],
                stg_sems.at[ping * 2 + j],
            )
            dma.start()
            dmas[(ping, j)] = dma

        own_scale = jnp.exp(m0 - mcol) * inv_z
        emit(0, 0, ea * own_scale, my_pos * v_loc)
        emit(0, 1, eb * own_scale, my_pos * v_loc + v_half)

        for h in range(N_DEV - 1):
            ping = (h + 1) % 2
            ring_r[h].wait_recv()
            if h < N_DEV - 2:
                ring_r.append(mk_ring(comm_r, send_r, recv_r, h + 1, right))
                ring_r[h + 1].start()
            ring_l[h].wait_recv()
            if h < N_DEV - 2:
                ring_l.append(mk_ring(comm_l, send_l, recv_l, h + 1, left))
                ring_l[h + 1].start()
            ring_r[h].wait_send()
            ring_l[h].wait_send()

            for j in (0, 1):
                if (ping, j) in dmas:
                    dmas[(ping, j)].wait()

            origin_r = (my_pos - h - 1) % N_DEV
            origin_l = (my_pos + h + 1) % N_DEV
            fin_r = jnp.exp(
                comm_r[h + 1, :, :].astype(jnp.float32) - mcol) * inv_z
            emit(ping, 0, fin_r, origin_r * v_loc)
            fin_l = jnp.exp(
                comm_l[h + 1, :, :].astype(jnp.float32) - mcol) * inv_z
            emit(ping, 1, fin_l, origin_l * v_loc + v_half)

        for ping in (0, 1):
            for j in (0, 1):
                if (ping, j) in dmas:
                    dmas[(ping, j)].wait()

    return pl.pallas_call(
        body,
        out_shape=jax.ShapeDtypeStruct((t, N_DEV * v_loc), jnp.float32),
        in_specs=[
            pl.BlockSpec(memory_space=pl.ANY),
            pl.BlockSpec(memory_space=pl.ANY),
        ],
        out_specs=pl.BlockSpec(memory_space=pl.ANY),
        scratch_shapes=[
            pltpu.VMEM((N_DEV, t, v_half), jnp.bfloat16),
            pltpu.VMEM((N_DEV, t, v_half), jnp.bfloat16),
            pltpu.VMEM((N_DEV, 2, t, 128), jnp.float32),
            pltpu.VMEM((2, 2, t, v_half), jnp.float32),
            pltpu.VMEM((t, d), jnp.float32),
            pltpu.VMEM((d, v_loc), jnp.float32),
            pltpu.SemaphoreType.DMA((N_DEV - 1,)),
            pltpu.SemaphoreType.DMA((N_DEV - 1,)),
            pltpu.SemaphoreType.DMA((N_DEV - 1,)),
            pltpu.SemaphoreType.DMA((N_DEV - 1,)),
            pltpu.SemaphoreType.DMA((3,)),
            pltpu.SemaphoreType.DMA((N_DEV,)),
            pltpu.SemaphoreType.DMA((4,)),
            pltpu.SemaphoreType.DMA((3,)),
        ],
        compiler_params=pltpu.CompilerParams(
            collective_id=0,
            vmem_limit_bytes=100 * 1024 * 1024,
        ),
    )(x, W)


# device time: 66792 ns/iter; 1.5321x vs baseline; 1.0013x over previous
import jax
import jax.numpy as jnp
from jax import lax
from jax.experimental import pallas as pl
from jax.experimental.pallas import tpu as pltpu

N_DEV = 4


def kernel(x, W):
    t, d = x.shape
    _, v_loc = W.shape
    v_half = v_loc // 2

    def body(x_hbm, w_hbm, out_ref, comm_r, comm_l, stats_buf, stg,
             x_v, w_v,
             send_r, recv_r, send_l, recv_l,
             stats_send, stats_recv, stg_sems, in_sems):
        my_pos = lax.axis_index("i")
        left = (my_pos - 1) % N_DEV
        right = (my_pos + 1) % N_DEV
        diag = (my_pos + 2) % N_DEV

        dma_x = pltpu.make_async_copy(x_hbm, x_v, in_sems.at[0])
        dma_wa = pltpu.make_async_copy(
            w_hbm.at[:, pl.ds(0, v_half)],
            w_v.at[:, pl.ds(0, v_half)], in_sems.at[1])
        dma_wb = pltpu.make_async_copy(
            w_hbm.at[:, pl.ds(v_half, v_half)],
            w_v.at[:, pl.ds(v_half, v_half)], in_sems.at[2])
        dma_x.start()
        dma_wa.start()
        dma_wb.start()

        barrier_sem = pltpu.get_barrier_semaphore()
        for nbr in [left, right, diag]:
            pl.semaphore_signal(
                barrier_sem, inc=1,
                device_id=(nbr,), device_id_type=pl.DeviceIdType.MESH,
            )
        pl.semaphore_wait(barrier_sem, N_DEV - 1)

        def mk_ring(comm, sends, recvs, h, dev):
            return pltpu.make_async_remote_copy(
                src_ref=comm.at[h], dst_ref=comm.at[h + 1],
                send_sem=sends.at[h], recv_sem=recvs.at[h],
                device_id=(dev,), device_id_type=pl.DeviceIdType.MESH,
            )

        dma_x.wait()
        dma_wa.wait()
        logits_a = jnp.dot(x_v[:, :], w_v[:, pl.ds(0, v_half)],
                           preferred_element_type=jnp.float32)
        comm_r[0, :, :] = logits_a.astype(jnp.bfloat16)
        ring_r = [mk_ring(comm_r, send_r, recv_r, 0, right)]
        ring_r[0].start()

        dma_wb.wait()
        logits_b = jnp.dot(x_v[:, :], w_v[:, pl.ds(v_half, v_half)],
                           preferred_element_type=jnp.float32)
        comm_l[0, :, :] = logits_b.astype(jnp.bfloat16)
        ring_l = [mk_ring(comm_l, send_l, recv_l, 0, left)]
        ring_l[0].start()


        ma = jnp.max(logits_a, axis=-1, keepdims=True)
        mb = jnp.max(logits_b, axis=-1, keepdims=True)
        m0 = jnp.maximum(ma, mb)
        ea = jnp.exp(logits_a - m0)
        eb = jnp.exp(logits_b - m0)
        s0 = (jnp.sum(ea, axis=-1, keepdims=True)
              + jnp.sum(eb, axis=-1, keepdims=True))
        stats_buf[my_pos, 0, :, :] = jnp.broadcast_to(m0, (t, 128))
        stats_buf[my_pos, 1, :, :] = jnp.broadcast_to(s0, (t, 128))
        stats_rdmas = []
        for k, p in enumerate([left, right, diag]):
            r = pltpu.make_async_remote_copy(
                src_ref=stats_buf.at[my_pos], dst_ref=stats_buf.at[my_pos],
                send_sem=stats_send.at[k], recv_sem=stats_recv.at[my_pos],
                device_id=(p,), device_id_type=pl.DeviceIdType.MESH,
            )
            r.start()
            stats_rdmas.append(r)
        for p in [left, right, diag]:
            pltpu.make_async_remote_copy(
                src_ref=stats_buf.at[p], dst_ref=stats_buf.at[p],
                send_sem=stats_send.at[0], recv_sem=stats_recv.at[p],
                device_id=(p,), device_id_type=pl.DeviceIdType.MESH,
            ).wait_recv()
        for r in stats_rdmas:
            r.wait_send()

        ms = [stats_buf[c, 0, :, :] for c in range(N_DEV)]
        ss = [stats_buf[c, 1, :, :] for c in range(N_DEV)]
        m = ms[0]
        for mc in ms[1:]:
            m = jnp.maximum(m, mc)
        z = ss[0] * jnp.exp(ms[0] - m)
        for mc, sc in zip(ms[1:], ss[1:]):
            z = z + sc * jnp.exp(mc - m)
        mcol = m[:, 0:1]
        inv_z = 1.0 / z[:, 0:1]

        dmas = {}

        def emit(ping, j, piece_final, start):
            stg[ping, j, :, :] = piece_final
            dma = pltpu.make_async_copy(
                stg.at[ping, j],
                out_ref.at[:, pl.ds(start, v_half)],
                stg_sems.at[ping * 2 + j],
            )
            dma.start()
            dmas[(ping, j)] = dma

        own_scale = jnp.exp(m0 - mcol) * inv_z
        emit(0, 0, ea * own_scale, my_pos * v_loc)
        emit(0, 1, eb * own_scale, my_pos * v_loc + v_half)

        v_q = v_half // 2

        def mk_sub(comm, sends, recvs, q, dev):
            return pltpu.make_async_remote_copy(
                src_ref=comm.at[2, :, pl.ds(q * v_q, v_q)],
                dst_ref=comm.at[3, :, pl.ds(q * v_q, v_q)],
                send_sem=sends.at[2 + q], recv_sem=recvs.at[2 + q],
                device_id=(dev,), device_id_type=pl.DeviceIdType.MESH,
            )

        sub_r = sub_l = None
        for h in (0, 1):
            ping = (h + 1) % 2
            ring_r[h].wait_recv()
            if h == 0:
                ring_r.append(mk_ring(comm_r, send_r, recv_r, 1, right))
                ring_r[1].start()
            else:
                sub_r = [mk_sub(comm_r, send_r, recv_r, q, right)
                         for q in (0, 1)]
                sub_r[0].start()
                sub_r[1].start()
            ring_l[h].wait_recv()
            if h == 0:
                ring_l.append(mk_ring(comm_l, send_l, recv_l, 1, left))
                ring_l[1].start()
            else:
                sub_l = [mk_sub(comm_l, send_l, recv_l, q, left)
                         for q in (0, 1)]
                sub_l[0].start()
                sub_l[1].start()
            ring_r[h].wait_send()
            ring_l[h].wait_send()

            for j in (0, 1):
                if (ping, j) in dmas:
                    dmas[(ping, j)].wait()

            origin_r = (my_pos - h - 1) % N_DEV
            origin_l = (my_pos + h + 1) % N_DEV
            fin_r = jnp.exp(
                comm_r[h + 1, :, :].astype(jnp.float32) - mcol) * inv_z
            emit(ping, 0, fin_r, origin_r * v_loc)
            fin_l = jnp.exp(
                comm_l[h + 1, :, :].astype(jnp.float32) - mcol) * inv_z
            emit(ping, 1, fin_l, origin_l * v_loc + v_half)

        for j in (0, 1):
            if (1, j) in dmas:
                dmas[(1, j)].wait()
        origin_r = (my_pos - 3) % N_DEV
        origin_l = (my_pos + 3) % N_DEV
        for q in (0, 1):
            sub_r[q].wait_recv()
            stg[1, 0, :, pl.ds(q * v_q, v_q)] = jnp.exp(
                comm_r[3, :, pl.ds(q * v_q, v_q)].astype(jnp.float32)
                - mcol) * inv_z
            sub_l[q].wait_recv()
            stg[1, 1, :, pl.ds(q * v_q, v_q)] = jnp.exp(
                comm_l[3, :, pl.ds(q * v_q, v_q)].astype(jnp.float32)
                - mcol) * inv_z
        for rd in sub_r + sub_l:
            rd.wait_send()
        for j, start in ((0, origin_r * v_loc), (1, origin_l * v_loc + v_half)):
            dma = pltpu.make_async_copy(
                stg.at[1, j], out_ref.at[:, pl.ds(start, v_half)],
                stg_sems.at[2 + j],
            )
            dma.start()
            dmas[(1, j)] = dma

        for ping in (0, 1):
            for j in (0, 1):
                if (ping, j) in dmas:
                    dmas[(ping, j)].wait()

    return pl.pallas_call(
        body,
        out_shape=jax.ShapeDtypeStruct((t, N_DEV * v_loc), jnp.float32),
        in_specs=[
            pl.BlockSpec(memory_space=pl.ANY),
            pl.BlockSpec(memory_space=pl.ANY),
        ],
        out_specs=pl.BlockSpec(memory_space=pl.ANY),
        scratch_shapes=[
            pltpu.VMEM((N_DEV, t, v_half), jnp.bfloat16),
            pltpu.VMEM((N_DEV, t, v_half), jnp.bfloat16),
            pltpu.VMEM((N_DEV, 2, t, 128), jnp.float32),
            pltpu.VMEM((2, 2, t, v_half), jnp.float32),
            pltpu.VMEM((t, d), jnp.float32),
            pltpu.VMEM((d, v_loc), jnp.float32),
            pltpu.SemaphoreType.DMA((4,)),
            pltpu.SemaphoreType.DMA((4,)),
            pltpu.SemaphoreType.DMA((4,)),
            pltpu.SemaphoreType.DMA((4,)),
            pltpu.SemaphoreType.DMA((3,)),
            pltpu.SemaphoreType.DMA((N_DEV,)),
            pltpu.SemaphoreType.DMA((4,)),
            pltpu.SemaphoreType.DMA((3,)),
        ],
        compiler_params=pltpu.CompilerParams(
            collective_id=0,
            vmem_limit_bytes=100 * 1024 * 1024,
        ),
    )(x, W)
